# Initial kernel scaffold; baseline (speedup 1.0000x reference)
import jax
import jax.numpy as jnp
from jax.experimental import pallas as pl


def kernel(attn_bias, spatial_pos, attn_edge_type, spatial_pos_table, edge_table, virtual_dist):
    raise NotImplementedError("write your pallas kernel here")


# speedup vs baseline: 7.7114x; 7.7114x over previous
import functools

import jax
import jax.numpy as jnp
from jax import lax
from jax.experimental import pallas as pl
from jax.experimental.pallas import tpu as pltpu
from jax.experimental.pallas import tpu_sc as plsc

B, N, H = 32, 128, 32
NP = N + 1
JP = 144
NG = JP // 16
VS = 512
VE = 1537

_mesh = plsc.VectorSubcoreMesh(core_axis_name="c", subcore_axis_name="s")


@functools.partial(
    pl.kernel,
    mesh=_mesh,
    compiler_params=pltpu.CompilerParams(use_tc_tiling_on_sc=False,
                                         needs_layout_passes=False),
    out_type=jax.ShapeDtypeStruct((B, H, NP, NP), jnp.float32),
    scratch_types=[
        pltpu.VMEM(((VS + 1) * H,), jnp.float32),
        pltpu.VMEM(((VE + 1) * H,), jnp.float32),
        pltpu.VMEM((JP,), jnp.int32),
        pltpu.VMEM((3, JP), jnp.int32),
        pltpu.VMEM((JP,), jnp.float32),
        pltpu.VMEM((H, NP), jnp.float32),
        pltpu.SemaphoreType.DMA,
    ],
)
def _graph_attn_bias_sc(sp_h, ed_h, spi_h, ei_h, bias_h, out_h,
                        sp_tab, ed_tab, spi, ei, brow, orow, sem):
    b = lax.axis_index("s") * 2 + lax.axis_index("c")
    pltpu.async_copy(sp_h, sp_tab, sem).wait()
    pltpu.async_copy(ed_h, ed_tab, sem).wait()

    def row_body(r, carry):
        c1 = pltpu.async_copy(spi_h.at[b, r], spi, sem)
        c2 = pltpu.async_copy(ei_h.at[b, :, r], ei, sem)
        c3 = pltpu.async_copy(bias_h.at[b, r], brow, sem)
        c1.wait()
        c2.wait()
        c3.wait()
        for jg in range(N // 16):
            sl = pl.ds(jg * 16, 16)
            a_sp = jnp.clip(spi[sl], 0, VS) * H
            a_e0 = jnp.clip(ei[0, sl], 0, VE) * H
            a_e1 = jnp.clip(ei[1, sl], 0, VE) * H
            a_e2 = jnp.clip(ei[2, sl], 0, VE) * H
            bv = brow[sl]

            def h_body(h, c, sl=sl, bv=bv):
                asp, a0, a1, a2 = c
                v = plsc.load_gather(sp_tab, [asp])
                e = (plsc.load_gather(ed_tab, [a0])
                     + plsc.load_gather(ed_tab, [a1])
                     + plsc.load_gather(ed_tab, [a2]))
                orow[h, sl] = bv + v + e * (1.0 / 3.0)
                return (asp + 1, a0 + 1, a1 + 1, a2 + 1)

            lax.fori_loop(0, H, h_body, (a_sp, a_e0, a_e1, a_e2))
        c128 = jnp.full((16,), N, jnp.int32)
        isp = jnp.clip(plsc.load_gather(spi, [c128]), 0, VS) * H
        ie0 = jnp.clip(plsc.load_gather(ei, [jnp.zeros((16,), jnp.int32),
                                             c128]), 0, VE) * H
        ie1 = jnp.clip(plsc.load_gather(ei, [jnp.ones((16,), jnp.int32),
                                             c128]), 0, VE) * H
        ie2 = jnp.clip(plsc.load_gather(ei, [jnp.full((16,), 2, jnp.int32),
                                             c128]), 0, VE) * H
        bvc = plsc.load_gather(brow, [c128])
        hv = jax.lax.iota(jnp.int32, 16)
        for g in range(H // 16):
            hg = hv + g * 16
            v = plsc.load_gather(sp_tab, [isp + hg])
            e = (plsc.load_gather(ed_tab, [ie0 + hg])
                 + plsc.load_gather(ed_tab, [ie1 + hg])
                 + plsc.load_gather(ed_tab, [ie2 + hg]))
            plsc.store_scatter(orow, [hg, c128], bvc + v + e * (1.0 / 3.0))
        pltpu.async_copy(orow, out_h.at[b, :, r, :], sem).wait()
        return carry

    lax.fori_loop(0, NP, row_body, 0)


def kernel(attn_bias, spatial_pos, attn_edge_type, spatial_pos_table,
           edge_table, virtual_dist):
    f32 = jnp.float32
    sp_aug = jnp.concatenate(
        [spatial_pos_table.astype(f32), virtual_dist.astype(f32).reshape(1, H)],
        axis=0).reshape(-1)
    ed_aug = jnp.concatenate(
        [edge_table.astype(f32), jnp.zeros((1, H), f32)], axis=0).reshape(-1)
    spi = spatial_pos.astype(jnp.int32)
    spi = jnp.pad(spi, ((0, 0), (1, 0), (1, 0)), constant_values=VS)
    spi = jnp.pad(spi, ((0, 0), (0, 0), (0, JP - NP)), constant_values=0)
    ei = attn_edge_type.astype(jnp.int32).transpose(0, 3, 1, 2)
    ei = jnp.pad(ei, ((0, 0), (0, 0), (1, 0), (1, 0)), constant_values=VE)
    ei = jnp.pad(ei, ((0, 0), (0, 0), (0, 0), (0, JP - NP)), constant_values=0)
    biasp = jnp.pad(attn_bias.astype(f32), ((0, 0), (0, 0), (0, JP - NP)))
    return _graph_attn_bias_sc(sp_aug, ed_aug, spi, ei, biasp)


# speedup vs baseline: 7.7640x; 1.0068x over previous
import functools

import jax
import jax.numpy as jnp
from jax import lax
from jax.experimental import pallas as pl
from jax.experimental.pallas import tpu as pltpu
from jax.experimental.pallas import tpu_sc as plsc

B, N, H = 32, 128, 32
NP = N + 1
JP = 144
NG = JP // 16
VS = 512
VE = 1537

_mesh = plsc.VectorSubcoreMesh(core_axis_name="c", subcore_axis_name="s")


@functools.partial(
    pl.kernel,
    mesh=_mesh,
    compiler_params=pltpu.CompilerParams(use_tc_tiling_on_sc=False,
                                         needs_layout_passes=False),
    out_type=jax.ShapeDtypeStruct((B, H, NP, NP), jnp.float32),
    scratch_types=[
        pltpu.VMEM(((VS + 1) * H,), jnp.float32),
        pltpu.VMEM(((VE + 1) * H,), jnp.float32),
        pltpu.VMEM((JP,), jnp.int32),
        pltpu.VMEM((3, JP), jnp.int32),
        pltpu.VMEM((JP,), jnp.float32),
        pltpu.VMEM((H, NP), jnp.float32),
        pltpu.SemaphoreType.DMA,
    ],
)
def _graph_attn_bias_sc(sp_h, ed_h, spi_h, ei_h, bias_h, out_h,
                        sp_tab, ed_tab, spi, ei, brow, orow, sem):
    b = lax.axis_index("s") * 2 + lax.axis_index("c")
    pltpu.async_copy(sp_h, sp_tab, sem).wait()
    pltpu.async_copy(ed_h, ed_tab, sem).wait()

    def row_body(r, carry):
        c1 = pltpu.async_copy(spi_h.at[b, r], spi, sem)
        c2 = pltpu.async_copy(ei_h.at[b, :, r], ei, sem)
        c3 = pltpu.async_copy(bias_h.at[b, r], brow, sem)
        c1.wait()
        c2.wait()
        c3.wait()
        for jg in range(N // 16):
            sl = pl.ds(jg * 16, 16)
            a_sp = jnp.clip(spi[sl], 0, VS) * H
            a_e0 = jnp.clip(ei[0, sl], 0, VE) * H
            a_e1 = jnp.clip(ei[1, sl], 0, VE) * H
            a_e2 = jnp.clip(ei[2, sl], 0, VE) * H
            bv = brow[sl]

            def h_body(h, c, sl=sl, bv=bv):
                asp, a0, a1, a2 = c
                v = plsc.load_gather(sp_tab, [asp])
                e = (plsc.load_gather(ed_tab, [a0])
                     + plsc.load_gather(ed_tab, [a1])
                     + plsc.load_gather(ed_tab, [a2]))
                orow[h, sl] = bv + v + e * (1.0 / 3.0)
                return (asp + 1, a0 + 1, a1 + 1, a2 + 1)

            lax.fori_loop(0, H, h_body, (a_sp, a_e0, a_e1, a_e2),
                          unroll=True)
        c128 = jnp.full((16,), N, jnp.int32)
        isp = jnp.clip(plsc.load_gather(spi, [c128]), 0, VS) * H
        ie0 = jnp.clip(plsc.load_gather(ei, [jnp.zeros((16,), jnp.int32),
                                             c128]), 0, VE) * H
        ie1 = jnp.clip(plsc.load_gather(ei, [jnp.ones((16,), jnp.int32),
                                             c128]), 0, VE) * H
        ie2 = jnp.clip(plsc.load_gather(ei, [jnp.full((16,), 2, jnp.int32),
                                             c128]), 0, VE) * H
        bvc = plsc.load_gather(brow, [c128])
        hv = jax.lax.iota(jnp.int32, 16)
        for g in range(H // 16):
            hg = hv + g * 16
            v = plsc.load_gather(sp_tab, [isp + hg])
            e = (plsc.load_gather(ed_tab, [ie0 + hg])
                 + plsc.load_gather(ed_tab, [ie1 + hg])
                 + plsc.load_gather(ed_tab, [ie2 + hg]))
            plsc.store_scatter(orow, [hg, c128], bvc + v + e * (1.0 / 3.0))
        pltpu.async_copy(orow, out_h.at[b, :, r, :], sem).wait()
        return carry

    lax.fori_loop(0, NP, row_body, 0)


def kernel(attn_bias, spatial_pos, attn_edge_type, spatial_pos_table,
           edge_table, virtual_dist):
    f32 = jnp.float32
    sp_aug = jnp.concatenate(
        [spatial_pos_table.astype(f32), virtual_dist.astype(f32).reshape(1, H)],
        axis=0).reshape(-1)
    ed_aug = jnp.concatenate(
        [edge_table.astype(f32), jnp.zeros((1, H), f32)], axis=0).reshape(-1)
    spi = spatial_pos.astype(jnp.int32)
    spi = jnp.pad(spi, ((0, 0), (1, 0), (1, 0)), constant_values=VS)
    spi = jnp.pad(spi, ((0, 0), (0, 0), (0, JP - NP)), constant_values=0)
    ei = attn_edge_type.astype(jnp.int32).transpose(0, 3, 1, 2)
    ei = jnp.pad(ei, ((0, 0), (0, 0), (1, 0), (1, 0)), constant_values=VE)
    ei = jnp.pad(ei, ((0, 0), (0, 0), (0, 0), (0, JP - NP)), constant_values=0)
    biasp = jnp.pad(attn_bias.astype(f32), ((0, 0), (0, 0), (0, JP - NP)))
    return _graph_attn_bias_sc(sp_aug, ed_aug, spi, ei, biasp)
